# baseline (speedup 1.0000x reference)
import functools

import jax
import jax.numpy as jnp
from jax import lax
from jax.experimental import pallas as pl
from jax.experimental.pallas import tpu as pltpu


def _compute_block(z_ref, wexp_ref, wsq_ref, i, scale):
    z = z_ref[i].astype(jnp.float32)
    zw = jnp.dot(z, wexp_ref[...], preferred_element_type=jnp.float32)
    ss = jnp.dot(z * z, wsq_ref[...], preferred_element_type=jnp.float32)
    inv = lax.rsqrt(jnp.maximum(ss, 1e-24)) * scale
    f = (zw * inv).astype(jnp.bfloat16)
    gram = lax.dot_general(
        f, f,
        dimension_numbers=(((1,), (1,)), ((), ())),
        preferred_element_type=jnp.float32,
    )
    return jnp.maximum(gram, 0.0)


def _ring_kernel(z_ref, wexp_ref, wsq_ref, o_ref, ring, sems, *,
                 scale, b_per_core, n_slots):
    core = pl.program_id(0)
    j = pl.program_id(1)
    b = core * b_per_core + j
    slot = lax.rem(j, n_slots)

    @pl.when(j >= n_slots)
    def _():
        pltpu.make_async_copy(ring.at[slot], ring.at[slot],
                              sems.at[slot]).wait()

    ring[pl.ds(slot, 1)] = _compute_block(z_ref, wexp_ref, wsq_ref, 0,
                                          scale)[None]
    pltpu.make_async_copy(ring.at[slot], o_ref.at[b], sems.at[slot]).start()

    @pl.when(j == b_per_core - 1)
    def _():
        for s in range(n_slots):
            pltpu.make_async_copy(ring.at[s], ring.at[s], sems.at[s]).wait()


def _emitter_kernel(z_ref, wexp_ref, wsq_ref, o_ref, *, scale):
    for i in range(z_ref.shape[0]):
        o_ref[i] = _compute_block(z_ref, wexp_ref, wsq_ref, i, scale)


def _round_up(x, m):
    return ((x + m - 1) // m) * m


def kernel(Z, w):
    B, N, D = Z.shape
    H, Dw = w.shape
    assert D == Dw, "w feature dim must match Z feature dim"
    HD = H * D

    n_pad = _round_up(N, 8) if N <= 128 else _round_up(N, 128)
    if n_pad != N:
        Z = jnp.pad(Z, ((0, 0), (0, n_pad - N), (0, 0)))

    w32 = w.astype(jnp.float32)
    eye = jnp.eye(D, dtype=jnp.float32)
    w_exp = (eye[None, :, :] * w32[:, None, :]).transpose(1, 0, 2).reshape(D, HD)
    w_sq = jnp.repeat((w32 * w32).T, D, axis=1)

    scale = 1.0 / (H ** 0.5)
    block_bytes = n_pad * n_pad * 4

    if B % 2 == 0 and block_bytes * 6 <= 40 * 1024 * 1024:
        b_per_core = B // 2
        n_slots = min(4, b_per_core)
        ring_kernel = functools.partial(
            _ring_kernel, scale=scale, b_per_core=b_per_core,
            n_slots=n_slots)
        att = pl.pallas_call(
            ring_kernel,
            out_shape=jax.ShapeDtypeStruct((B, n_pad, n_pad), jnp.float32),
            grid=(2, b_per_core),
            in_specs=[
                pl.BlockSpec((1, n_pad, D),
                             lambda c, j, b2=b_per_core: (c * b2 + j, 0, 0)),
                pl.BlockSpec((D, HD), lambda c, j: (0, 0)),
                pl.BlockSpec((D, HD), lambda c, j: (0, 0)),
            ],
            out_specs=pl.BlockSpec(memory_space=pl.ANY),
            scratch_shapes=[
                pltpu.VMEM((n_slots, n_pad, n_pad), jnp.float32),
                pltpu.SemaphoreType.DMA((n_slots,)),
            ],
            compiler_params=pltpu.CompilerParams(
                dimension_semantics=("parallel", "arbitrary"),
                vmem_limit_bytes=56 * 1024 * 1024,
            ),
        )(Z, w_exp, w_sq)
    else:
        emitter = functools.partial(_emitter_kernel, scale=scale)
        att = pl.pallas_call(
            emitter,
            out_shape=jax.ShapeDtypeStruct((B, n_pad, n_pad), jnp.float32),
            grid=(B,),
            in_specs=[
                pl.BlockSpec((1, n_pad, D), lambda b: (b, 0, 0)),
                pl.BlockSpec((D, HD), lambda b: (0, 0)),
                pl.BlockSpec((D, HD), lambda b: (0, 0)),
            ],
            out_specs=pl.BlockSpec((1, n_pad, n_pad), lambda b: (b, 0, 0)),
            compiler_params=pltpu.CompilerParams(
                dimension_semantics=("parallel",),
                vmem_limit_bytes=48 * 1024 * 1024,
            ),
        )(Z, w_exp, w_sq)

    if n_pad != N:
        att = att[:, :N, :N]
    return att
